# Initial kernel scaffold; baseline (speedup 1.0000x reference)
import jax
import jax.numpy as jnp
from jax.experimental import pallas as pl


def kernel(data, wemb, pemb, semb):
    raise NotImplementedError("write your pallas kernel here")


# speedup vs baseline: 7.9415x; 7.9415x over previous
import functools

import jax
import jax.numpy as jnp
from jax import lax
from jax.experimental import pallas as pl
from jax.experimental.pallas import tpu as pltpu
from jax.experimental.pallas import tpu_sc as plsc


def _combine_body(w_ref, p_ref, s_ref, o_ref):
    o_ref[...] = (w_ref[...] + p_ref[...]) + s_ref[...]


def _combine_tables(wemb, pemb, semb):
    v, d = wemb.shape
    flat = v * d
    w2 = wemb.reshape(flat // 128, 128)
    p2 = pemb.reshape(flat // 128, 128)
    s2 = semb.reshape(flat // 128, 128)
    rows = w2.shape[0]
    block = 2000
    out = pl.pallas_call(
        _combine_body,
        grid=(rows // block,),
        in_specs=[pl.BlockSpec((block, 128), lambda i: (i, 0))] * 3,
        out_specs=pl.BlockSpec((block, 128), lambda i: (i, 0)),
        out_shape=jax.ShapeDtypeStruct((rows, 128), jnp.float32),
    )(w2, p2, s2)
    return out.reshape(v, d)


def _sc_gather(table, idx_flat):
    v, d = table.shape
    b = idx_flat.shape[0]
    info = plsc.get_sparse_core_info()
    nw = info.num_cores * info.num_subcores
    b_per_w = b // nw
    chunk = 128
    n_chunks = b_per_w // chunk
    mesh = plsc.VectorSubcoreMesh(core_axis_name="c", subcore_axis_name="s")

    @functools.partial(
        pl.kernel,
        mesh=mesh,
        out_type=jax.ShapeDtypeStruct((b, d), jnp.float32),
        scratch_types=[
            pltpu.VMEM((chunk,), jnp.int32),
            pltpu.VMEM((chunk, d), jnp.float32),
            pltpu.SemaphoreType.DMA,
        ],
        compiler_params=pltpu.CompilerParams(use_tc_tiling_on_sc=False),
    )
    def gather_kernel(table_hbm, idx_hbm, out_hbm, idx_v, rows_v, gsem):
        wid = lax.axis_index("s") * info.num_cores + lax.axis_index("c")
        base = wid * b_per_w

        def body(g, carry):
            off = pl.multiple_of(base + g * chunk, chunk)
            pltpu.sync_copy(idx_hbm.at[pl.ds(off, chunk)], idx_v)
            pltpu.async_copy(table_hbm.at[idx_v], rows_v, gsem).wait()
            pltpu.sync_copy(rows_v, out_hbm.at[pl.ds(off, chunk)])
            return carry

        lax.fori_loop(0, n_chunks, body, 0)

    return gather_kernel(table, idx_flat)


def kernel(data, wemb, pemb, semb):
    batch, seq = data.shape
    d = wemb.shape[1]
    combined = _combine_tables(wemb, pemb, semb)
    idx_flat = data.reshape(-1).astype(jnp.int32)
    out = _sc_gather(combined, idx_flat)
    return out.reshape(batch, seq, d)


# speedup vs baseline: 9.2044x; 1.1590x over previous
import functools

import jax
import jax.numpy as jnp
from jax import lax
from jax.experimental import pallas as pl
from jax.experimental.pallas import tpu as pltpu
from jax.experimental.pallas import tpu_sc as plsc


def _combine_body(w_ref, p_ref, s_ref, o_ref):
    o_ref[...] = (w_ref[...] + p_ref[...]) + s_ref[...]


def _combine_tables(wemb, pemb, semb):
    v, d = wemb.shape
    flat = v * d
    w2 = wemb.reshape(flat // 128, 128)
    p2 = pemb.reshape(flat // 128, 128)
    s2 = semb.reshape(flat // 128, 128)
    rows = w2.shape[0]
    block = 2000
    out = pl.pallas_call(
        _combine_body,
        grid=(rows // block,),
        in_specs=[pl.BlockSpec((block, 128), lambda i: (i, 0))] * 3,
        out_specs=pl.BlockSpec((block, 128), lambda i: (i, 0)),
        out_shape=jax.ShapeDtypeStruct((rows, 128), jnp.float32),
    )(w2, p2, s2)
    return out.reshape(v, d)


def _sc_gather(table, idx2d):
    v, d = table.shape
    n_rows, chunk = idx2d.shape
    info = plsc.get_sparse_core_info()
    nw = info.num_cores * info.num_subcores
    n_chunks = n_rows // nw
    b_per_w = n_chunks * chunk
    b = n_rows * chunk
    mesh = plsc.VectorSubcoreMesh(core_axis_name="c", subcore_axis_name="s")

    @functools.partial(
        pl.kernel,
        mesh=mesh,
        out_type=jax.ShapeDtypeStruct((b, d), jnp.float32),
        scratch_types=[
            pltpu.VMEM((n_chunks, chunk), jnp.int32),
            pltpu.VMEM((2, chunk, d), jnp.float32),
            pltpu.SemaphoreType.DMA,
            pltpu.SemaphoreType.DMA((2,)),
            pltpu.SemaphoreType.DMA((2,)),
        ],
        compiler_params=pltpu.CompilerParams(use_tc_tiling_on_sc=False),
    )
    def gather_kernel(table_hbm, idx_hbm, out_hbm, idx_v, rows_v, isem,
                      gsem, osem):
        wid = lax.axis_index("s") * info.num_cores + lax.axis_index("c")
        row_base = wid * n_chunks
        out_base = wid * b_per_w

        pltpu.async_copy(idx_hbm.at[pl.ds(row_base, n_chunks)], idx_v,
                         isem).wait()

        def start_gather(g, slot):
            pltpu.async_copy(table_hbm.at[idx_v.at[g]], rows_v.at[slot],
                             gsem.at[slot])

        def wait_gather(slot):
            pltpu.make_async_copy(table_hbm.at[idx_v.at[0]], rows_v.at[slot],
                                  gsem.at[slot]).wait()

        def start_write(g, slot):
            off = pl.multiple_of(out_base + g * chunk, chunk)
            pltpu.async_copy(rows_v.at[slot], out_hbm.at[pl.ds(off, chunk)],
                             osem.at[slot])

        def wait_write(slot):
            off0 = pl.multiple_of(out_base, chunk)
            pltpu.make_async_copy(rows_v.at[slot],
                                  out_hbm.at[pl.ds(off0, chunk)],
                                  osem.at[slot]).wait()

        n_pairs = n_chunks // 2
        start_gather(0, 0)

        def body(i, carry):
            g0 = i * 2
            wait_gather(0)
            start_write(g0, 0)

            @pl.when(i >= 1)
            def _():
                wait_write(1)

            start_gather(g0 + 1, 1)
            wait_gather(1)
            start_write(g0 + 1, 1)

            @pl.when(i < n_pairs - 1)
            def _():
                wait_write(0)
                start_gather(g0 + 2, 0)

            return carry

        lax.fori_loop(0, n_pairs, body, 0)
        wait_write(0)
        wait_write(1)

    return gather_kernel(table, idx2d)


def kernel(data, wemb, pemb, semb):
    batch, seq = data.shape
    d = wemb.shape[1]
    combined = _combine_tables(wemb, pemb, semb)
    idx2d = data.reshape(-1, 128).astype(jnp.int32)
    out = _sc_gather(combined, idx2d)
    return out.reshape(batch, seq, d)
